# baseline (speedup 1.0000x reference)
import functools

import jax
import jax.numpy as jnp
from jax import lax
from jax.experimental import pallas as pl
from jax.experimental.pallas import tpu as pltpu
from jax.experimental.pallas import tpu_sc as plsc

SEQ_LEN = 200
EMBED_DIM = 64
IMG_FEAT = 2048
OUT_DIM = 1000

_NC = 2
_NS = 16
_NW = _NC * _NS
_ROWS_PER_W = 8
_ACTIVE_W = SEQ_LEN // _ROWS_PER_W

_ROW_BLK = 200
_GRID = OUT_DIM // _ROW_BLK


def _sc_gather_sum_body(idx_hbm, table_hbm, out_hbm, idx_v, rows_v, acc_v, sem):
    wid = lax.axis_index("s") * _NC + lax.axis_index("c")

    @pl.when(wid < _ACTIVE_W)
    def _():
        base = wid * _ROWS_PER_W
        pltpu.sync_copy(idx_hbm.at[pl.ds(base, _ROWS_PER_W)], idx_v)
        pltpu.async_copy(table_hbm.at[idx_v], rows_v, sem).wait()
        for d in range(EMBED_DIM // 16):
            s = rows_v[0, pl.ds(d * 16, 16)]
            for r in range(1, _ROWS_PER_W):
                s = s + rows_v[r, pl.ds(d * 16, 16)]
            acc_v[pl.ds(d * 16, 16)] = s
        pltpu.sync_copy(acc_v, out_hbm.at[wid])

    @pl.when(wid >= _ACTIVE_W)
    def _():
        for d in range(EMBED_DIM // 16):
            acc_v[pl.ds(d * 16, 16)] = jnp.zeros((16,), jnp.float32)
        pltpu.sync_copy(acc_v, out_hbm.at[wid])


@functools.cache
def _sc_gather_sum():
    return pl.kernel(
        _sc_gather_sum_body,
        out_type=jax.ShapeDtypeStruct((_NW, EMBED_DIM), jnp.float32),
        mesh=plsc.VectorSubcoreMesh(core_axis_name="c", subcore_axis_name="s"),
        scratch_types=[
            pltpu.VMEM((_ROWS_PER_W,), jnp.int32),
            pltpu.VMEM((_ROWS_PER_W, EMBED_DIM), jnp.float32),
            pltpu.VMEM((EMBED_DIM,), jnp.float32),
            pltpu.SemaphoreType.DMA,
        ],
        compiler_params=pltpu.CompilerParams(use_tc_tiling_on_sc=False),
    )


def _tc_body(img_ref, wi_ref, wt_ref, part_ref, b_ref, out_ref, score_vmem):
    i = pl.program_id(0)
    blk = lax.dot_general(
        img_ref[...], wi_ref[...],
        (((1,), (1,)), ((), ())),
        preferred_element_type=jnp.float32,
    )
    score_vmem[pl.ds(i * _ROW_BLK, _ROW_BLK), :] = blk

    @pl.when(i == _GRID - 1)
    def _():
        se = jnp.sum(part_ref[...], axis=0, keepdims=True)
        t = jnp.sum(se * wt_ref[...]) + b_ref[0, 0]
        s = score_vmem[...] + t
        m = jnp.max(s)
        e = jnp.exp(s - m)
        out_ref[...] = e / jnp.sum(e)


_tc_matvec_softmax = pl.pallas_call(
    _tc_body,
    grid=(_GRID,),
    in_specs=[
        pl.BlockSpec((_ROW_BLK, IMG_FEAT), lambda i: (i, 0)),
        pl.BlockSpec((1, IMG_FEAT), lambda i: (0, 0)),
        pl.BlockSpec((1, EMBED_DIM), lambda i: (0, 0)),
        pl.BlockSpec((_NW, EMBED_DIM), lambda i: (0, 0)),
        pl.BlockSpec((1, 1), lambda i: (0, 0)),
    ],
    out_specs=pl.BlockSpec((OUT_DIM, 1), lambda i: (0, 0)),
    out_shape=jax.ShapeDtypeStruct((OUT_DIM, 1), jnp.float32),
    scratch_shapes=[pltpu.VMEM((OUT_DIM, 1), jnp.float32)],
)


def kernel(text_input, image_input, emb_table, W, b):
    idx = text_input.reshape(SEQ_LEN).astype(jnp.int32)
    partials = _sc_gather_sum()(idx, emb_table)
    img = image_input.reshape(OUT_DIM, IMG_FEAT)
    wt = W[:, :EMBED_DIM]
    wi = W[:, EMBED_DIM:]
    probs = _tc_matvec_softmax(img, wi, wt, partials, b.reshape(1, 1))
    return probs.reshape(1, OUT_DIM)


# speedup vs baseline: 1.7333x; 1.7333x over previous
import functools

import jax
import jax.numpy as jnp
from jax import lax
from jax.experimental import pallas as pl
from jax.experimental.pallas import tpu as pltpu
from jax.experimental.pallas import tpu_sc as plsc

VOCAB = 1000000
SEQ_LEN = 200
EMBED_DIM = 64
IMG_FEAT = 2048
OUT_DIM = 1000

_NC = 2
_NS = 16
_NW = _NC * _NS

_ROW_BLK = 200
_GRID = OUT_DIM // _ROW_BLK


_ROWS_PER_W = 8
_ACTIVE_W = SEQ_LEN // _ROWS_PER_W


def _sc_gather_sum_body(idx_hbm, table_hbm, out_hbm, idx_v, rows_v, acc_v, sem):
    wid = lax.axis_index("s") * _NC + lax.axis_index("c")

    @pl.when(wid < _ACTIVE_W)
    def _():
        base = wid * _ROWS_PER_W
        pltpu.sync_copy(idx_hbm.at[pl.ds(base, _ROWS_PER_W)],
                        idx_v.at[pl.ds(0, _ROWS_PER_W)])
        iv = idx_v[...]
        copies = []
        for k in range(_ROWS_PER_W):
            r = iv[k]
            copies.append(
                pltpu.async_copy(table_hbm.at[r], rows_v.at[k], sem))
        for c in copies:
            c.wait()
        for d in range(EMBED_DIM // 16):
            s = rows_v[0, pl.ds(d * 16, 16)]
            for k in range(1, _ROWS_PER_W):
                s = s + rows_v[k, pl.ds(d * 16, 16)]
            acc_v[pl.ds(d * 16, 16)] = s
        pltpu.sync_copy(acc_v, out_hbm.at[wid])

    @pl.when(wid >= _ACTIVE_W)
    def _():
        for d in range(EMBED_DIM // 16):
            acc_v[pl.ds(d * 16, 16)] = jnp.zeros((16,), jnp.float32)
        pltpu.sync_copy(acc_v, out_hbm.at[wid])


@functools.cache
def _sc_gather_sum():
    return pl.kernel(
        _sc_gather_sum_body,
        out_type=jax.ShapeDtypeStruct((_NW, EMBED_DIM), jnp.float32),
        mesh=plsc.VectorSubcoreMesh(core_axis_name="c", subcore_axis_name="s"),
        scratch_types=[
            pltpu.VMEM((16,), jnp.int32),
            pltpu.VMEM((_ROWS_PER_W, EMBED_DIM), jnp.float32),
            pltpu.VMEM((EMBED_DIM,), jnp.float32),
            pltpu.SemaphoreType.DMA,
        ],
        compiler_params=pltpu.CompilerParams(needs_layout_passes=False),
    )


def _tc_body(img_ref, wi_ref, wt_ref, part_ref, b_ref, out_ref, score_vmem):
    i = pl.program_id(0)
    blk = lax.dot_general(
        img_ref[...], wi_ref[...],
        (((1,), (1,)), ((), ())),
        preferred_element_type=jnp.float32,
    )
    score_vmem[pl.ds(i * _ROW_BLK, _ROW_BLK), :] = blk

    @pl.when(i == _GRID - 1)
    def _():
        se = jnp.sum(part_ref[...], axis=0, keepdims=True)
        t = jnp.sum(se * wt_ref[...]) + b_ref[0, 0]
        s = score_vmem[...] + t
        m = jnp.max(s)
        e = jnp.exp(s - m)
        out_ref[...] = e / jnp.sum(e)


_tc_matvec_softmax = pl.pallas_call(
    _tc_body,
    grid=(_GRID,),
    in_specs=[
        pl.BlockSpec((_ROW_BLK, IMG_FEAT), lambda i: (i, 0)),
        pl.BlockSpec((1, IMG_FEAT), lambda i: (0, 0)),
        pl.BlockSpec((1, EMBED_DIM), lambda i: (0, 0)),
        pl.BlockSpec((_NW, EMBED_DIM), lambda i: (0, 0)),
        pl.BlockSpec((1, 1), lambda i: (0, 0)),
    ],
    out_specs=pl.BlockSpec((OUT_DIM, 1), lambda i: (0, 0)),
    out_shape=jax.ShapeDtypeStruct((OUT_DIM, 1), jnp.float32),
    scratch_shapes=[pltpu.VMEM((OUT_DIM, 1), jnp.float32)],
)


def kernel(text_input, image_input, emb_table, W, b):
    idx = text_input.reshape(SEQ_LEN).astype(jnp.int32)
    partials = _sc_gather_sum()(idx, emb_table)
    img = image_input.reshape(OUT_DIM, IMG_FEAT)
    wt = W[:, :EMBED_DIM]
    wi = W[:, EMBED_DIM:]
    probs = _tc_matvec_softmax(img, wi, wt, partials, b.reshape(1, 1))
    return probs.reshape(1, OUT_DIM)
